# Initial kernel scaffold; baseline (speedup 1.0000x reference)
import jax
import jax.numpy as jnp
from jax.experimental import pallas as pl


def kernel(image, adj_s, W0, b0, W1, b1, W2, b2, Wc1, bc1, Wc2, bc2, Wc3, bc3):
    raise NotImplementedError("write your pallas kernel here")


# speedup vs baseline: 6482.7468x; 6482.7468x over previous
import jax
import jax.numpy as jnp
from jax.experimental import pallas as pl

N = 2048
D_IN = 128
D_H = 128
NC = 3

_TN = (((0,), (0,)), ((), ()))


def _gcn_kernel(a_ref, x_ref, w0_ref, b0_ref, w1_ref, b1_ref, w2_ref, b2_ref,
                wc1_ref, bc1_ref, wc2_ref, bc2_ref, wc3_ref, bc3_ref, o_ref):
    A = a_ref[...]
    ones = jnp.ones((N, 1), dtype=jnp.float32)
    deg = jax.lax.dot_general(A, ones, _TN,
                              preferred_element_type=jnp.float32) + 1.0
    d = jax.lax.rsqrt(deg)

    x = x_ref[...]
    for w_ref, b_ref in ((w0_ref, b0_ref), (w1_ref, b1_ref), (w2_ref, b2_ref)):
        y = jnp.dot(x, w_ref[...], preferred_element_type=jnp.float32)
        z = y * d
        agg = jax.lax.dot_general(A, z, _TN,
                                  preferred_element_type=jnp.float32) + z
        x = jnp.maximum(agg * d + b_ref[...], 0.0)

    x_max = jnp.max(x, axis=0, keepdims=True)
    x_mean = jnp.mean(x, axis=0, keepdims=True)
    g = jnp.concatenate([x_max, x_mean], axis=1)
    h = jnp.maximum(jnp.dot(g, wc1_ref[...],
                            preferred_element_type=jnp.float32) + bc1_ref[...], 0.0)
    h = jnp.maximum(jnp.dot(h, wc2_ref[...],
                            preferred_element_type=jnp.float32) + bc2_ref[...], 0.0)
    o_ref[...] = jnp.dot(h, wc3_ref[...],
                         preferred_element_type=jnp.float32) + bc3_ref[...]


@jax.jit
def _run(image, adj_s, W0, b0, W1, b1, W2, b2, Wc1, bc1, Wc2, bc2, Wc3, bc3):
    out = pl.pallas_call(
        _gcn_kernel,
        out_shape=jax.ShapeDtypeStruct((1, NC), jnp.float32),
    )(adj_s, image,
      W0, b0.reshape(1, -1), W1, b1.reshape(1, -1), W2, b2.reshape(1, -1),
      Wc1, bc1.reshape(1, -1), Wc2, bc2.reshape(1, -1),
      Wc3, bc3.reshape(1, -1))
    return out.reshape(NC)


def kernel(image, adj_s, W0, b0, W1, b1, W2, b2, Wc1, bc1, Wc2, bc2, Wc3, bc3):
    return _run(image, adj_s, W0, b0, W1, b1, W2, b2,
                Wc1, bc1, Wc2, bc2, Wc3, bc3)


# speedup vs baseline: 7651.2856x; 1.1803x over previous
import jax
import jax.numpy as jnp
from jax.experimental import pallas as pl
from jax.experimental.pallas import tpu as pltpu

N = 2048
D_IN = 128
D_H = 128
NC = 3
NCHUNK = 8
CH = N // NCHUNK

_TN = (((0,), (0,)), ((), ()))


def _gcn_kernel(a_hbm, x_ref, w0_ref, b0_ref, w1_ref, b1_ref, w2_ref, b2_ref,
                wc1_ref, bc1_ref, wc2_ref, bc2_ref, wc3_ref, bc3_ref, o_ref,
                a_vmem, sems):
    for k in range(NCHUNK):
        pltpu.make_async_copy(
            a_hbm.at[pl.ds(k * CH, CH), :],
            a_vmem.at[pl.ds(k * CH, CH), :],
            sems.at[k]).start()

    x = x_ref[...]
    y0 = jnp.dot(x, w0_ref[...], preferred_element_type=jnp.float32)

    colsum = jnp.full((1, N), 1.0, dtype=jnp.float32)
    for k in range(NCHUNK):
        pltpu.make_async_copy(
            a_hbm.at[pl.ds(k * CH, CH), :],
            a_vmem.at[pl.ds(k * CH, CH), :],
            sems.at[k]).wait()
        colsum = colsum + jnp.sum(a_vmem[pl.ds(k * CH, CH), :], axis=0,
                                  keepdims=True)
    d = jax.lax.rsqrt(jnp.transpose(colsum, (1, 0)))

    A = a_vmem[...]
    y = y0
    for li, (w_ref, b_ref) in enumerate(
            ((w0_ref, b0_ref), (w1_ref, b1_ref), (w2_ref, b2_ref))):
        if li > 0:
            y = jnp.dot(x, w_ref[...], preferred_element_type=jnp.float32)
        z = y * d
        agg = jax.lax.dot_general(A, z, _TN,
                                  preferred_element_type=jnp.float32) + z
        x = jnp.maximum(agg * d + b_ref[...], 0.0)

    x_max = jnp.max(x, axis=0, keepdims=True)
    x_mean = jnp.mean(x, axis=0, keepdims=True)
    g = jnp.concatenate([x_max, x_mean], axis=1)
    h = jnp.maximum(jnp.dot(g, wc1_ref[...],
                            preferred_element_type=jnp.float32) + bc1_ref[...], 0.0)
    h = jnp.maximum(jnp.dot(h, wc2_ref[...],
                            preferred_element_type=jnp.float32) + bc2_ref[...], 0.0)
    o_ref[...] = jnp.dot(h, wc3_ref[...],
                         preferred_element_type=jnp.float32) + bc3_ref[...]


@jax.jit
def _run(image, adj_s, W0, b0, W1, b1, W2, b2, Wc1, bc1, Wc2, bc2, Wc3, bc3):
    vmem = pl.BlockSpec(memory_space=pl.ANY)
    out = pl.pallas_call(
        _gcn_kernel,
        out_shape=jax.ShapeDtypeStruct((1, NC), jnp.float32),
        in_specs=[vmem] + [pl.BlockSpec(memory_space=pltpu.MemorySpace.VMEM)] * 13,
        scratch_shapes=[
            pltpu.VMEM((N, N), jnp.float32),
            pltpu.SemaphoreType.DMA((NCHUNK,)),
        ],
    )(adj_s, image,
      W0, b0.reshape(1, -1), W1, b1.reshape(1, -1), W2, b2.reshape(1, -1),
      Wc1, bc1.reshape(1, -1), Wc2, bc2.reshape(1, -1),
      Wc3, bc3.reshape(1, -1))
    return out.reshape(NC)


def kernel(image, adj_s, W0, b0, W1, b1, W2, b2, Wc1, bc1, Wc2, bc2, Wc3, bc3):
    return _run(image, adj_s, W0, b0, W1, b1, W2, b2,
                Wc1, bc1, Wc2, bc2, Wc3, bc3)


# speedup vs baseline: 7659.2672x; 1.0010x over previous
import jax
import jax.numpy as jnp
from jax.experimental import pallas as pl
from jax.experimental.pallas import tpu as pltpu

N = 2048
D_IN = 128
D_H = 128
NC = 3
NCHUNK = 8
CH = N // NCHUNK

_TN = (((0,), (0,)), ((), ()))


def _gcn_kernel(a_hbm, x_ref, w0_ref, b0_ref, w1_ref, b1_ref, w2_ref, b2_ref,
                wc1_ref, bc1_ref, wc2_ref, bc2_ref, wc3_ref, bc3_ref, o_ref,
                a_vmem, a_bf, sems):
    for k in range(NCHUNK):
        pltpu.make_async_copy(
            a_hbm.at[pl.ds(k * CH, CH), :],
            a_vmem.at[pl.ds(k * CH, CH), :],
            sems.at[k]).start()

    x = x_ref[...]
    y0 = jnp.dot(x, w0_ref[...], preferred_element_type=jnp.float32)

    colsum = jnp.full((1, N), 1.0, dtype=jnp.float32)
    for k in range(NCHUNK):
        pltpu.make_async_copy(
            a_hbm.at[pl.ds(k * CH, CH), :],
            a_vmem.at[pl.ds(k * CH, CH), :],
            sems.at[k]).wait()
        chunk = a_vmem[pl.ds(k * CH, CH), :]
        colsum = colsum + jnp.sum(chunk, axis=0, keepdims=True)
        a_bf[pl.ds(k * CH, CH), :] = chunk.astype(jnp.bfloat16)
    d = jax.lax.rsqrt(jnp.transpose(colsum, (1, 0)))

    A = a_bf[...]
    y = y0
    for li, (w_ref, b_ref) in enumerate(
            ((w0_ref, b0_ref), (w1_ref, b1_ref), (w2_ref, b2_ref))):
        if li > 0:
            y = jnp.dot(x, w_ref[...], preferred_element_type=jnp.float32)
        z = y * d
        agg = jax.lax.dot_general(A, z.astype(jnp.bfloat16), _TN,
                                  preferred_element_type=jnp.float32) + z
        x = jnp.maximum(agg * d + b_ref[...], 0.0)

    x_max = jnp.max(x, axis=0, keepdims=True)
    x_mean = jnp.mean(x, axis=0, keepdims=True)
    g = jnp.concatenate([x_max, x_mean], axis=1)
    h = jnp.maximum(jnp.dot(g, wc1_ref[...],
                            preferred_element_type=jnp.float32) + bc1_ref[...], 0.0)
    h = jnp.maximum(jnp.dot(h, wc2_ref[...],
                            preferred_element_type=jnp.float32) + bc2_ref[...], 0.0)
    o_ref[...] = jnp.dot(h, wc3_ref[...],
                         preferred_element_type=jnp.float32) + bc3_ref[...]


@jax.jit
def _run(image, adj_s, W0, b0, W1, b1, W2, b2, Wc1, bc1, Wc2, bc2, Wc3, bc3):
    vmem = pl.BlockSpec(memory_space=pl.ANY)
    out = pl.pallas_call(
        _gcn_kernel,
        out_shape=jax.ShapeDtypeStruct((1, NC), jnp.float32),
        in_specs=[vmem] + [pl.BlockSpec(memory_space=pltpu.MemorySpace.VMEM)] * 13,
        scratch_shapes=[
            pltpu.VMEM((N, N), jnp.float32),
            pltpu.VMEM((N, N), jnp.bfloat16),
            pltpu.SemaphoreType.DMA((NCHUNK,)),
        ],
    )(adj_s, image,
      W0, b0.reshape(1, -1), W1, b1.reshape(1, -1), W2, b2.reshape(1, -1),
      Wc1, bc1.reshape(1, -1), Wc2, bc2.reshape(1, -1),
      Wc3, bc3.reshape(1, -1))
    return out.reshape(NC)


def kernel(image, adj_s, W0, b0, W1, b1, W2, b2, Wc1, bc1, Wc2, bc2, Wc3, bc3):
    return _run(image, adj_s, W0, b0, W1, b1, W2, b2,
                Wc1, bc1, Wc2, bc2, Wc3, bc3)
